# baseline (speedup 1.0000x reference)
import jax
import jax.numpy as jnp
from jax import lax
from jax.experimental import pallas as pl
from jax.experimental.pallas import tpu as pltpu

BATCH = 16384
LENGTH = 1024
WIDTH = 2 * LENGTH
BT = 256


def _body(x_ref, a_ref, b_ref, o_ref):
    x = x_ref[...]
    lane = lax.broadcasted_iota(jnp.int32, x.shape, 1) & 7
    xl = pltpu.roll(x, WIDTH - 2, 1)
    xr = pltpu.roll(x, 2, 1)
    take_l = jnp.logical_or(lane == 2, lane == 3)
    take_r = jnp.logical_or(lane == 4, lane == 5)
    y = jnp.where(take_l, xl, jnp.where(take_r, xr, x))
    ys = jnp.where((lane & 1) == 0, pltpu.roll(y, WIDTH - 1, 1),
                   pltpu.roll(y, 1, 1))
    a = a_ref[...]
    b = b_ref[...]
    o_ref[...] = a * y + b * ys


def kernel(x, forward_indices, crossings):
    del forward_indices
    xf = x.reshape(BATCH, WIDTH)
    cr = crossings[:, 0]
    ci = crossings[:, 1]
    a = jnp.stack([cr, cr], axis=-1).reshape(1, WIDTH)
    b = jnp.stack([-ci, ci], axis=-1).reshape(1, WIDTH)

    out = pl.pallas_call(
        _body,
        grid=(BATCH // BT,),
        in_specs=[
            pl.BlockSpec((BT, WIDTH), lambda i: (i, 0)),
            pl.BlockSpec((1, WIDTH), lambda i: (0, 0)),
            pl.BlockSpec((1, WIDTH), lambda i: (0, 0)),
        ],
        out_specs=pl.BlockSpec((BT, WIDTH), lambda i: (i, 0)),
        out_shape=jax.ShapeDtypeStruct((BATCH, WIDTH), jnp.float32),
    )(xf, a, b)
    return out.reshape(BATCH, LENGTH, 2)


# speedup vs baseline: 1.0057x; 1.0057x over previous
import jax
import jax.numpy as jnp
from jax import lax
from jax.experimental import pallas as pl
from jax.experimental.pallas import tpu as pltpu

BATCH = 16384
LENGTH = 1024
WIDTH = 2 * LENGTH
BT = 1024


def _body(x_ref, a_ref, b_ref, o_ref):
    x = x_ref[...]
    lane = lax.broadcasted_iota(jnp.int32, x.shape, 1) & 7
    xl = pltpu.roll(x, WIDTH - 2, 1)
    xr = pltpu.roll(x, 2, 1)
    take_l = jnp.logical_or(lane == 2, lane == 3)
    take_r = jnp.logical_or(lane == 4, lane == 5)
    y = jnp.where(take_l, xl, jnp.where(take_r, xr, x))
    ys = jnp.where((lane & 1) == 0, pltpu.roll(y, WIDTH - 1, 1),
                   pltpu.roll(y, 1, 1))
    a = a_ref[...]
    b = b_ref[...]
    o_ref[...] = a * y + b * ys


def kernel(x, forward_indices, crossings):
    del forward_indices
    xf = x.reshape(BATCH, WIDTH)
    cr = crossings[:, 0]
    ci = crossings[:, 1]
    a = jnp.stack([cr, cr], axis=-1).reshape(1, WIDTH)
    b = jnp.stack([-ci, ci], axis=-1).reshape(1, WIDTH)

    out = pl.pallas_call(
        _body,
        grid=(BATCH // BT,),
        in_specs=[
            pl.BlockSpec((BT, WIDTH), lambda i: (i, 0)),
            pl.BlockSpec((1, WIDTH), lambda i: (0, 0)),
            pl.BlockSpec((1, WIDTH), lambda i: (0, 0)),
        ],
        out_specs=pl.BlockSpec((BT, WIDTH), lambda i: (i, 0)),
        out_shape=jax.ShapeDtypeStruct((BATCH, WIDTH), jnp.float32),
    )(xf, a, b)
    return out.reshape(BATCH, LENGTH, 2)
